# baseline (speedup 1.0000x reference)
import jax
import jax.numpy as jnp
from jax.experimental import pallas as pl

B = 1024
D = 32

def _const_kernel(out_ref):
    out_ref[...] = jnp.full(out_ref.shape, 0.5, jnp.float32)

def kernel(x, table, linear_weights):
    out = pl.pallas_call(
        _const_kernel,
        grid=(64,),
        out_specs=pl.BlockSpec((16, B, D), lambda i: (i, 0, 0)),
        out_shape=jax.ShapeDtypeStruct((B, B, D), jnp.float32),
    )()
    return out


# speedup vs baseline: 4.8467x; 4.8467x over previous
import jax
import jax.numpy as jnp
from jax.experimental import pallas as pl

B = 1024
F = 100
V = 100
D = 32

BI = 128
BI3 = 16


def _stats_kernel(x_ref, table_ref, lw_ref, cross_ref, lin_ref):
    x = x_ref[...]
    xf = x.astype(jnp.float32)
    lw = lw_ref[...]
    lin_ref[...] = 0.5 * jnp.sum(xf * lw, axis=1, keepdims=True)

    vals = jax.lax.broadcasted_iota(jnp.int32, (1, 1, V), 2)
    cmp = (x[:, :, None] == vals).astype(jnp.float32)
    counts = jnp.sum(cmp, axis=1)
    t = table_ref[...]
    cw = counts[:, :, None] * t[None, :, :]
    s = jnp.sum(cw, axis=1)
    ss = jnp.sum(cw * t[None, :, :], axis=1)
    cross_ref[...] = 0.25 * (s * s - ss)

def _outer_sigmoid_kernel(cross_ref, lin_ref, out_ref):
    h = cross_ref[...][:, :, None] + lin_ref[...][None, :, :]
    out_ref[...] = 0.5 * jnp.tanh(h) + 0.5


def kernel(x, table, linear_weights):
    lw2 = linear_weights.reshape(1, F)

    cross, lin = pl.pallas_call(
        _stats_kernel,
        grid=(B // BI,),
        in_specs=[
            pl.BlockSpec((BI, F), lambda i: (i, 0)),
            pl.BlockSpec((V, D), lambda i: (0, 0)),
            pl.BlockSpec((1, F), lambda i: (0, 0)),
        ],
        out_specs=[
            pl.BlockSpec((BI, D), lambda i: (i, 0)),
            pl.BlockSpec((BI, 1), lambda i: (i, 0)),
        ],
        out_shape=[
            jax.ShapeDtypeStruct((B, D), jnp.float32),
            jax.ShapeDtypeStruct((B, 1), jnp.float32),
        ],
    )(x, table, lw2)

    lin_row = lin.reshape(1, B)

    out3 = pl.pallas_call(
        _outer_sigmoid_kernel,
        grid=(B // BI3,),
        in_specs=[
            pl.BlockSpec((BI3, D), lambda i: (i, 0)),
            pl.BlockSpec((1, B), lambda i: (0, 0)),
        ],
        out_specs=pl.BlockSpec((BI3, D, B), lambda i: (i, 0, 0)),
        out_shape=jax.ShapeDtypeStruct((B, D, B), jnp.float32),
    )(cross, lin_row)

    return jnp.swapaxes(out3, 1, 2)


# speedup vs baseline: 6.2865x; 1.2971x over previous
import jax
import jax.numpy as jnp
from jax.experimental import pallas as pl

B = 1024
F = 100
V = 100
D = 32

BI = 128
BI3 = 16


def _stats_kernel(xt_ref, tt_ref, lw_ref, cross_ref, lin_ref):
    xt = xt_ref[...]
    lw = lw_ref[...]
    lin_ref[...] = 0.5 * jnp.sum(xt.astype(jnp.float32) * lw, axis=0,
                                 keepdims=True)

    tt = tt_ref[...]
    s = jnp.zeros((D, xt.shape[1]), jnp.float32)
    ss = jnp.zeros((D, xt.shape[1]), jnp.float32)
    for f in range(F):
        idx = jnp.broadcast_to(xt[f:f + 1, :], (D, xt.shape[1]))
        rows = jnp.take_along_axis(tt, idx, axis=1)
        s = s + rows
        ss = ss + rows * rows
    cross_ref[...] = 0.25 * (s * s - ss)


def _outer_sigmoid_kernel(cross_ref, lin_ref, out_ref):
    h = cross_ref[...][:, :, None] + lin_ref[...][None, :, :]
    out_ref[...] = 0.5 * jnp.tanh(h) + 0.5


def kernel(x, table, linear_weights):
    xt = x.T
    tt = table.T
    lw2 = linear_weights.reshape(F, 1)

    cross_t, lin_row = pl.pallas_call(
        _stats_kernel,
        grid=(1,),
        in_specs=[
            pl.BlockSpec((F, B), lambda i: (0, 0)),
            pl.BlockSpec((D, V), lambda i: (0, 0)),
            pl.BlockSpec((F, 1), lambda i: (0, 0)),
        ],
        out_specs=[
            pl.BlockSpec((D, B), lambda i: (0, 0)),
            pl.BlockSpec((1, B), lambda i: (0, 0)),
        ],
        out_shape=[
            jax.ShapeDtypeStruct((D, B), jnp.float32),
            jax.ShapeDtypeStruct((1, B), jnp.float32),
        ],
    )(xt, tt, lw2)

    cross = cross_t.T

    out3 = pl.pallas_call(
        _outer_sigmoid_kernel,
        grid=(B // BI3,),
        in_specs=[
            pl.BlockSpec((BI3, D), lambda i: (i, 0)),
            pl.BlockSpec((1, B), lambda i: (0, 0)),
        ],
        out_specs=pl.BlockSpec((BI3, D, B), lambda i: (i, 0, 0)),
        out_shape=jax.ShapeDtypeStruct((B, D, B), jnp.float32),
    )(cross, lin_row)

    return jnp.swapaxes(out3, 1, 2)


# speedup vs baseline: 8.0639x; 1.2827x over previous
import jax
import jax.numpy as jnp
from jax.experimental import pallas as pl

B = 1024
F = 100
V = 100
D = 32

BI = 128
BI3 = 32


def _stats_kernel(xt_ref, tt_ref, lw_ref, cross_ref, lin_ref):
    xt = xt_ref[...]
    lw = lw_ref[...]
    lin_ref[...] = 0.5 * jnp.sum(xt.astype(jnp.float32) * lw, axis=0,
                                 keepdims=True)

    tt = tt_ref[...]
    s = jnp.zeros((D, xt.shape[1]), jnp.float32)
    ss = jnp.zeros((D, xt.shape[1]), jnp.float32)
    for f in range(F):
        idx = jnp.broadcast_to(xt[f:f + 1, :], (D, xt.shape[1]))
        rows = jnp.take_along_axis(tt, idx, axis=1)
        s = s + rows
        ss = ss + rows * rows
    cross_ref[...] = 0.25 * (s * s - ss)


def _outer_sigmoid_kernel(cross_ref, lin_ref, out_ref):
    h = cross_ref[...][:, :, None] + lin_ref[...][None, :, :]
    out_ref[...] = 0.5 * jnp.tanh(h) + 0.5


def kernel(x, table, linear_weights):
    xt = x.T
    tt = table.T
    lw2 = linear_weights.reshape(F, 1)

    cross_t, lin_row = pl.pallas_call(
        _stats_kernel,
        grid=(1,),
        in_specs=[
            pl.BlockSpec((F, B), lambda i: (0, 0)),
            pl.BlockSpec((D, V), lambda i: (0, 0)),
            pl.BlockSpec((F, 1), lambda i: (0, 0)),
        ],
        out_specs=[
            pl.BlockSpec((D, B), lambda i: (0, 0)),
            pl.BlockSpec((1, B), lambda i: (0, 0)),
        ],
        out_shape=[
            jax.ShapeDtypeStruct((D, B), jnp.float32),
            jax.ShapeDtypeStruct((1, B), jnp.float32),
        ],
    )(xt, tt, lw2)

    cross = cross_t.T

    out3 = pl.pallas_call(
        _outer_sigmoid_kernel,
        grid=(B // BI3,),
        in_specs=[
            pl.BlockSpec((BI3, D), lambda i: (i, 0)),
            pl.BlockSpec((1, B), lambda i: (0, 0)),
        ],
        out_specs=pl.BlockSpec((BI3, D, B), lambda i: (i, 0, 0)),
        out_shape=jax.ShapeDtypeStruct((B, D, B), jnp.float32),
    )(cross, lin_row)

    return jnp.swapaxes(out3, 1, 2)


# speedup vs baseline: 8.1676x; 1.0129x over previous
import jax
import jax.numpy as jnp
from jax.experimental import pallas as pl

B = 1024
F = 100
V = 100
D = 32

BI = 128
BI3 = 64


def _stats_kernel(xt_ref, tt_ref, lw_ref, cross_ref, lin_ref):
    xt = xt_ref[...]
    lw = lw_ref[...]
    lin_ref[...] = 0.5 * jnp.sum(xt.astype(jnp.float32) * lw, axis=0,
                                 keepdims=True)

    tt = tt_ref[...]
    s = jnp.zeros((D, xt.shape[1]), jnp.float32)
    ss = jnp.zeros((D, xt.shape[1]), jnp.float32)
    for f in range(F):
        idx = jnp.broadcast_to(xt[f:f + 1, :], (D, xt.shape[1]))
        rows = jnp.take_along_axis(tt, idx, axis=1)
        s = s + rows
        ss = ss + rows * rows
    cross_ref[...] = 0.25 * (s * s - ss)


def _outer_sigmoid_kernel(cross_ref, lin_ref, out_ref):
    h = cross_ref[...][:, :, None] + lin_ref[...][None, :, :]
    out_ref[...] = 0.5 * jnp.tanh(h) + 0.5


def kernel(x, table, linear_weights):
    xt = x.T
    tt = table.T
    lw2 = linear_weights.reshape(F, 1)

    cross_t, lin_row = pl.pallas_call(
        _stats_kernel,
        grid=(1,),
        in_specs=[
            pl.BlockSpec((F, B), lambda i: (0, 0)),
            pl.BlockSpec((D, V), lambda i: (0, 0)),
            pl.BlockSpec((F, 1), lambda i: (0, 0)),
        ],
        out_specs=[
            pl.BlockSpec((D, B), lambda i: (0, 0)),
            pl.BlockSpec((1, B), lambda i: (0, 0)),
        ],
        out_shape=[
            jax.ShapeDtypeStruct((D, B), jnp.float32),
            jax.ShapeDtypeStruct((1, B), jnp.float32),
        ],
    )(xt, tt, lw2)

    cross = cross_t.T

    out3 = pl.pallas_call(
        _outer_sigmoid_kernel,
        grid=(B // BI3,),
        in_specs=[
            pl.BlockSpec((BI3, D), lambda i: (i, 0)),
            pl.BlockSpec((1, B), lambda i: (0, 0)),
        ],
        out_specs=pl.BlockSpec((BI3, D, B), lambda i: (i, 0, 0)),
        out_shape=jax.ShapeDtypeStruct((B, D, B), jnp.float32),
    )(cross, lin_row)

    return jnp.swapaxes(out3, 1, 2)


# speedup vs baseline: 8.4258x; 1.0316x over previous
import jax
import jax.numpy as jnp
from jax.experimental import pallas as pl

B = 1024
F = 100
V = 100
D = 32

BI = 128
BI3 = 64


def _stats_kernel(xt_ref, tt_ref, lw_ref, cross_ref, lin_ref):
    xt = xt_ref[...]
    lw = lw_ref[...]
    lin_ref[...] = 0.5 * jnp.sum(xt.astype(jnp.float32) * lw, axis=0,
                                 keepdims=True)

    tt = tt_ref[...]
    s = jnp.zeros((D, xt.shape[1]), jnp.float32)
    ss = jnp.zeros((D, xt.shape[1]), jnp.float32)
    for f in range(F):
        idx = jnp.broadcast_to(xt[f:f + 1, :], (D, xt.shape[1]))
        rows = jnp.take_along_axis(tt, idx, axis=1)
        s = s + rows
        ss = ss + rows * rows
    cross_ref[...] = (0.25 * (s * s - ss)).T


def _outer_sigmoid_kernel(cross_ref, lin_ref, out_ref):
    h = cross_ref[...][:, :, None] + lin_ref[...][None, :, :]
    out_ref[...] = 0.5 * jnp.tanh(h) + 0.5


def kernel(x, table, linear_weights):
    xt = x.T
    tt = table.T
    lw2 = linear_weights.reshape(F, 1)

    cross, lin_row = pl.pallas_call(
        _stats_kernel,
        grid=(1,),
        in_specs=[
            pl.BlockSpec((F, B), lambda i: (0, 0)),
            pl.BlockSpec((D, V), lambda i: (0, 0)),
            pl.BlockSpec((F, 1), lambda i: (0, 0)),
        ],
        out_specs=[
            pl.BlockSpec((B, D), lambda i: (0, 0)),
            pl.BlockSpec((1, B), lambda i: (0, 0)),
        ],
        out_shape=[
            jax.ShapeDtypeStruct((B, D), jnp.float32),
            jax.ShapeDtypeStruct((1, B), jnp.float32),
        ],
    )(xt, tt, lw2)


    out3 = pl.pallas_call(
        _outer_sigmoid_kernel,
        grid=(B // BI3,),
        in_specs=[
            pl.BlockSpec((BI3, D), lambda i: (i, 0)),
            pl.BlockSpec((1, B), lambda i: (0, 0)),
        ],
        out_specs=pl.BlockSpec((BI3, D, B), lambda i: (i, 0, 0)),
        out_shape=jax.ShapeDtypeStruct((B, D, B), jnp.float32),
    )(cross, lin_row)

    return jnp.swapaxes(out3, 1, 2)


# speedup vs baseline: 8.6574x; 1.0275x over previous
import jax
import jax.numpy as jnp
from jax.experimental import pallas as pl
from jax.experimental.pallas import tpu as pltpu

B = 1024
F = 100
V = 100
D = 32

BI3 = 64


def _fm_kernel(xt_ref, tt_ref, lw_ref, out_ref, cross_s, lin_s):
    i = pl.program_id(0)

    @pl.when(i == 0)
    def _stats():
        xt = xt_ref[...]
        lw = lw_ref[...]
        lin_s[...] = 0.5 * jnp.sum(xt.astype(jnp.float32) * lw, axis=0,
                                   keepdims=True)
        tt = tt_ref[...]
        s = jnp.zeros((D, B), jnp.float32)
        ss = jnp.zeros((D, B), jnp.float32)
        for f in range(F):
            idx = jnp.broadcast_to(xt[f:f + 1, :], (D, B))
            rows = jnp.take_along_axis(tt, idx, axis=1)
            s = s + rows
            ss = ss + rows * rows
        cross_s[...] = (0.25 * (s * s - ss)).T

    cross_blk = cross_s[pl.ds(i * BI3, BI3), :]
    h = cross_blk[:, :, None] + lin_s[...][None, :, :]
    out_ref[...] = 0.5 * jnp.tanh(h) + 0.5


def kernel(x, table, linear_weights):
    xt = x.T
    tt = table.T
    lw2 = linear_weights.reshape(F, 1)

    out3 = pl.pallas_call(
        _fm_kernel,
        grid=(B // BI3,),
        in_specs=[
            pl.BlockSpec((F, B), lambda i: (0, 0)),
            pl.BlockSpec((D, V), lambda i: (0, 0)),
            pl.BlockSpec((F, 1), lambda i: (0, 0)),
        ],
        out_specs=pl.BlockSpec((BI3, D, B), lambda i: (i, 0, 0)),
        out_shape=jax.ShapeDtypeStruct((B, D, B), jnp.float32),
        scratch_shapes=[
            pltpu.VMEM((B, D), jnp.float32),
            pltpu.VMEM((1, B), jnp.float32),
        ],
    )(xt, tt, lw2)

    return jnp.swapaxes(out3, 1, 2)
